# baseline (speedup 1.0000x reference)
import functools

import jax
import jax.numpy as jnp
from jax import lax
from jax.experimental import pallas as pl
from jax.experimental.pallas import tpu as pltpu
from jax.experimental.pallas import tpu_sc as plsc

_N_EMB = 8192
_DIM = 32
_BETA = 0.25
_N = 16384

_R = 1024
_KC = 512

_NC = 2
_NS = 16
_NW = _NC * _NS
_BPW = _N // _NW
_CHUNK = 128
_NCHUNK = _BPW // _CHUNK


def _argmin_body(xt_ref, emb_ref, idx_ref, loss_ref, e2_ref, idsf_ref):
    i = pl.program_id(0)

    @pl.when(i == 0)
    def _prep():
        emb0 = emb_ref[...]
        e2_ref[...] = jnp.sum(emb0 * emb0, axis=0, keepdims=True)
        ids0 = lax.broadcasted_iota(jnp.int32, (1, _N_EMB), 1)
        idsf_ref[...] = ids0.astype(jnp.float32)
        loss_ref[...] = jnp.zeros_like(loss_ref)

    xt = xt_ref[0]
    sim2 = lax.dot_general(
        xt + xt,
        emb_ref[...],
        (((0,), (0,)), ((), ())),
        preferred_element_type=jnp.float32,
    )
    x2 = jnp.transpose(jnp.sum(xt * xt, axis=0, keepdims=True))
    cms = []
    cidxs = []
    for c in range(_N_EMB // _KC):
        lo, hi = c * _KC, (c + 1) * _KC
        sl = pl.ds(lo, _KC)
        dist = (x2 + e2_ref[:, sl]) - sim2[:, lo:hi]
        cm = jnp.min(dist, axis=1, keepdims=True)
        cidxf = jnp.min(
            jnp.where(dist == cm, idsf_ref[:, sl], jnp.float32(_N_EMB)),
            axis=1,
            keepdims=True,
        )
        cms.append(cm)
        cidxs.append(cidxf)
    cms = jnp.concatenate(cms, axis=1)
    cidxs = jnp.concatenate(cidxs, axis=1)
    m = jnp.min(cms, axis=1, keepdims=True)
    idxf = jnp.min(
        jnp.where(cms == m, cidxs, jnp.float32(_N_EMB)), axis=1
    )
    idx_ref[...] = idxf.astype(jnp.int32)
    loss_ref[...] += jnp.sum(m, keepdims=True)


def _argmin_call(xt3, embeddings):
    grid = _N // _R
    return pl.pallas_call(
        _argmin_body,
        grid=(grid,),
        in_specs=[
            pl.BlockSpec((1, _DIM, _R), lambda i: (i, 0, 0)),
            pl.BlockSpec((_DIM, _N_EMB), lambda i: (0, 0)),
        ],
        out_specs=[
            pl.BlockSpec((_R,), lambda i: (i,)),
            pl.BlockSpec((1, 1), lambda i: (0, 0)),
        ],
        out_shape=[
            jax.ShapeDtypeStruct((_N,), jnp.int32),
            jax.ShapeDtypeStruct((1, 1), jnp.float32),
        ],
        scratch_shapes=[
            pltpu.VMEM((1, _N_EMB), jnp.float32),
            pltpu.VMEM((1, _N_EMB), jnp.float32),
        ],
    )(xt3, embeddings)


@functools.lru_cache(maxsize=1)
def _sc_gather_kernel():
    mesh = plsc.VectorSubcoreMesh(core_axis_name="c", subcore_axis_name="s")

    @functools.partial(
        pl.kernel,
        mesh=mesh,
        out_type=jax.ShapeDtypeStruct((16, 1024, _DIM), jnp.float32),
        scratch_types=[
            pltpu.VMEM((_NCHUNK, _CHUNK), jnp.int32),
            pltpu.VMEM((_BPW, _DIM), jnp.float32),
            pltpu.SemaphoreType.DMA,
        ],
        compiler_params=pltpu.CompilerParams(use_tc_tiling_on_sc=False),
    )
    def _sc_gather(table_hbm, idx_hbm, out_hbm, idx_v, rows_v, sem):
        wid = lax.axis_index("s") * _NC + lax.axis_index("c")
        base = wid * _BPW
        pltpu.sync_copy(idx_hbm.at[wid], idx_v)
        copies = [
            pltpu.async_copy(
                table_hbm.at[idx_v.at[j]],
                rows_v.at[pl.ds(j * _CHUNK, _CHUNK)],
                sem,
            )
            for j in range(_NCHUNK)
        ]
        for c in copies:
            c.wait()
        pltpu.sync_copy(
            rows_v, out_hbm.at[wid // 2, pl.ds((wid % 2) * _BPW, _BPW)]
        )

    return _sc_gather


def kernel(x, embeddings):
    xt3 = x.transpose(0, 2, 1)
    idx, loss_sum = _argmin_call(xt3, embeddings)
    emb_t = embeddings.T
    out = _sc_gather_kernel()(emb_t, idx.reshape(_NW, _NCHUNK, _CHUNK))
    loss = (_BETA - 1.0) * (loss_sum[0, 0] / jnp.float32(x.size))
    return out, loss


# speedup vs baseline: 1.0670x; 1.0670x over previous
import functools

import jax
import jax.numpy as jnp
from jax import lax
from jax.experimental import pallas as pl
from jax.experimental.pallas import tpu as pltpu
from jax.experimental.pallas import tpu_sc as plsc

_N_EMB = 8192
_DIM = 32
_BETA = 0.25
_N = 16384

_R = 1024
_KC = 512

_NC = 2
_NS = 16
_NW = _NC * _NS
_BPW = _N // _NW
_CHUNK = 128
_NCHUNK = _BPW // _CHUNK


def _argmin_body(x_ref, emb_ref, idx_ref, loss_ref, e2_ref, idsf_ref):
    i = pl.program_id(0)

    @pl.when(i == 0)
    def _prep():
        emb0 = emb_ref[...]
        e2_ref[...] = jnp.sum(emb0 * emb0, axis=0, keepdims=True)
        ids0 = lax.broadcasted_iota(jnp.int32, (1, _N_EMB), 1)
        idsf_ref[...] = ids0.astype(jnp.float32)
        loss_ref[...] = jnp.zeros_like(loss_ref)

    x = x_ref[...]
    sim2 = jnp.dot(x + x, emb_ref[...], preferred_element_type=jnp.float32)
    x2 = jnp.sum(x * x, axis=1, keepdims=True)
    cms = []
    cidxs = []
    for c in range(_N_EMB // _KC):
        lo, hi = c * _KC, (c + 1) * _KC
        sl = pl.ds(lo, _KC)
        dist = (x2 + e2_ref[:, sl]) - sim2[:, lo:hi]
        cm = jnp.min(dist, axis=1, keepdims=True)
        cidxf = jnp.min(
            jnp.where(dist == cm, idsf_ref[:, sl], jnp.float32(_N_EMB)),
            axis=1,
            keepdims=True,
        )
        cms.append(cm)
        cidxs.append(cidxf)
    cms = jnp.concatenate(cms, axis=1)
    cidxs = jnp.concatenate(cidxs, axis=1)
    m = jnp.min(cms, axis=1, keepdims=True)
    idxf = jnp.min(
        jnp.where(cms == m, cidxs, jnp.float32(_N_EMB)), axis=1
    )
    idx_ref[...] = idxf.astype(jnp.int32)
    loss_ref[...] += jnp.sum(m, keepdims=True)


def _argmin_call(flat, embeddings):
    grid = _N // _R
    return pl.pallas_call(
        _argmin_body,
        grid=(grid,),
        in_specs=[
            pl.BlockSpec((_R, _DIM), lambda i: (i, 0)),
            pl.BlockSpec((_DIM, _N_EMB), lambda i: (0, 0)),
        ],
        out_specs=[
            pl.BlockSpec((_R,), lambda i: (i,)),
            pl.BlockSpec((1, 1), lambda i: (0, 0)),
        ],
        out_shape=[
            jax.ShapeDtypeStruct((_N,), jnp.int32),
            jax.ShapeDtypeStruct((1, 1), jnp.float32),
        ],
        scratch_shapes=[
            pltpu.VMEM((1, _N_EMB), jnp.float32),
            pltpu.VMEM((1, _N_EMB), jnp.float32),
        ],
    )(flat, embeddings)


@functools.lru_cache(maxsize=1)
def _sc_gather_kernel():
    mesh = plsc.VectorSubcoreMesh(core_axis_name="c", subcore_axis_name="s")

    @functools.partial(
        pl.kernel,
        mesh=mesh,
        out_type=jax.ShapeDtypeStruct((16, 1024, _DIM), jnp.float32),
        scratch_types=[
            pltpu.VMEM((_NCHUNK, _CHUNK), jnp.int32),
            pltpu.VMEM((_BPW, _DIM), jnp.float32),
            pltpu.SemaphoreType.DMA,
        ],
        compiler_params=pltpu.CompilerParams(use_tc_tiling_on_sc=False),
    )
    def _sc_gather(table_hbm, idx_hbm, out_hbm, idx_v, rows_v, sem):
        wid = lax.axis_index("s") * _NC + lax.axis_index("c")
        base = wid * _BPW
        pltpu.sync_copy(idx_hbm.at[wid], idx_v)
        copies = [
            pltpu.async_copy(
                table_hbm.at[idx_v.at[j]],
                rows_v.at[pl.ds(j * _CHUNK, _CHUNK)],
                sem,
            )
            for j in range(_NCHUNK)
        ]
        for c in copies:
            c.wait()
        pltpu.sync_copy(
            rows_v, out_hbm.at[wid // 2, pl.ds((wid % 2) * _BPW, _BPW)]
        )

    return _sc_gather


def kernel(x, embeddings):
    flat = x.reshape(_N, _DIM)
    idx, loss_sum = _argmin_call(flat, embeddings)
    emb_t = embeddings.T
    out = _sc_gather_kernel()(emb_t, idx.reshape(_NW, _NCHUNK, _CHUNK))
    loss = (_BETA - 1.0) * (loss_sum[0, 0] / jnp.float32(flat.size))
    return out, loss
